# baseline (speedup 1.0000x reference)
import functools

import jax
import jax.numpy as jnp
from jax import lax
from jax.experimental import pallas as pl
from jax.experimental.pallas import tpu as pltpu
from jax.experimental.pallas import tpu_sc as plsc

_D = 1024
_NL = 3
_B = 16
_L = 512
_CH = 32
_NCH = 8


def _sc_body(src_hbm, tgt_hbm, o0, o1, o2, o3, o4, o5, buf, zrow):
    cid = lax.axis_index("c")
    sid = lax.axis_index("s")
    wid = sid * 2 + cid
    b = wid // 2
    half = wid % 2

    jobs = (
        (src_hbm, 0, o0), (src_hbm, 1, o1), (src_hbm, 2, o2),
        (tgt_hbm, 0, o3), (tgt_hbm, 1, o4), (tgt_hbm, 2, o5),
    )

    zeros16 = jnp.zeros((16,), jnp.float32)
    for i in range(_D // 16):
        zrow[0, pl.ds(i * 16, 16)] = zeros16

    @pl.when(half == 0)
    def _():
        for _, _l, out_hbm in jobs:
            pltpu.sync_copy(zrow, out_hbm.at[b, pl.ds(0, 1), :])

    out_base = 1 + half * 255
    for e_hbm, l, out_hbm in jobs:
        for k in range(_NCH):
            out_start = out_base + k * _CH
            pltpu.sync_copy(
                e_hbm.at[b, pl.ds(out_start - 1, _CH), pl.ds(l * _D, _D)],
                buf,
            )
            pltpu.sync_copy(buf, out_hbm.at[b, pl.ds(out_start, _CH), :])


def kernel(elmo_src, elmo_tgt):
    mesh = plsc.VectorSubcoreMesh(core_axis_name="c", subcore_axis_name="s")
    out_struct = jax.ShapeDtypeStruct((_B, _L, _D), jnp.float32)
    kern = functools.partial(
        pl.kernel,
        out_type=[out_struct] * 6,
        mesh=mesh,
        scratch_types=[
            pltpu.VMEM((_CH, _D), jnp.float32),
            pltpu.VMEM((1, _D), jnp.float32),
        ],
        compiler_params=pltpu.CompilerParams(use_tc_tiling_on_sc=False),
    )(_sc_body)
    return tuple(kern(elmo_src, elmo_tgt))


# speedup vs baseline: 1.0487x; 1.0487x over previous
import functools

import jax
import jax.numpy as jnp
from jax import lax
from jax.experimental import pallas as pl
from jax.experimental.pallas import tpu as pltpu
from jax.experimental.pallas import tpu_sc as plsc

_D = 1024
_NL = 3
_B = 16
_L = 512
_CH = 32
_NCH = 8


def _sc_body(src_hbm, tgt_hbm, o0, o1, o2, o3, o4, o5,
             buf0, buf1, zrow, gsem0, gsem1, ssem0, ssem1):
    cid = lax.axis_index("c")
    sid = lax.axis_index("s")
    wid = sid * 2 + cid
    b = wid // 2
    half = wid % 2

    jobs = (
        (src_hbm, 0, o0), (src_hbm, 1, o1), (src_hbm, 2, o2),
        (tgt_hbm, 0, o3), (tgt_hbm, 1, o4), (tgt_hbm, 2, o5),
    )

    zeros16 = jnp.zeros((16,), jnp.float32)
    for i in range(_D // 16):
        zrow[0, pl.ds(i * 16, 16)] = zeros16

    @pl.when(half == 0)
    def _():
        for _, _l, out_hbm in jobs:
            pltpu.sync_copy(zrow, out_hbm.at[b, pl.ds(0, 1), :])

    out_base = 1 + half * 255
    bufs = (buf0, buf1)
    gsems = (gsem0, gsem1)
    ssems = (ssem0, ssem1)

    chunks = []
    for e_hbm, l, out_hbm in jobs:
        for k in range(_NCH):
            chunks.append((e_hbm, l, out_hbm, k))

    def gather(i):
        e_hbm, l, _, k = chunks[i]
        p = i % 2
        out_start = out_base + k * _CH
        return pltpu.async_copy(
            e_hbm.at[b, pl.ds(out_start - 1, _CH), pl.ds(l * _D, _D)],
            bufs[p], gsems[p])

    def scatter(i):
        _, _, out_hbm, k = chunks[i]
        p = i % 2
        out_start = out_base + k * _CH
        return pltpu.async_copy(
            bufs[p], out_hbm.at[b, pl.ds(out_start, _CH), :], ssems[p])

    n = len(chunks)
    g = [None, None]
    s = [None, None]
    g[0] = gather(0)
    for i in range(n):
        p = i % 2
        q = (i + 1) % 2
        if i + 1 < n:
            if s[q] is not None:
                s[q].wait()
                s[q] = None
            g[q] = gather(i + 1)
        g[p].wait()
        s[p] = scatter(i)
    for h in s:
        if h is not None:
            h.wait()


def kernel(elmo_src, elmo_tgt):
    mesh = plsc.VectorSubcoreMesh(core_axis_name="c", subcore_axis_name="s")
    out_struct = jax.ShapeDtypeStruct((_B, _L, _D), jnp.float32)
    kern = functools.partial(
        pl.kernel,
        out_type=[out_struct] * 6,
        mesh=mesh,
        scratch_types=[
            pltpu.VMEM((_CH, _D), jnp.float32),
            pltpu.VMEM((_CH, _D), jnp.float32),
            pltpu.VMEM((1, _D), jnp.float32),
            pltpu.SemaphoreType.DMA,
            pltpu.SemaphoreType.DMA,
            pltpu.SemaphoreType.DMA,
            pltpu.SemaphoreType.DMA,
        ],
        compiler_params=pltpu.CompilerParams(use_tc_tiling_on_sc=False),
    )(_sc_body)
    return tuple(kern(elmo_src, elmo_tgt))


# speedup vs baseline: 2.5056x; 2.3891x over previous
import functools

import jax
import jax.numpy as jnp
from jax import lax
from jax.experimental import pallas as pl
from jax.experimental.pallas import tpu as pltpu
from jax.experimental.pallas import tpu_sc as plsc

_D = 1024
_B = 16
_L = 512
_CH = 32
_NCH = 8


def _sc_body(src_hbm, tgt_hbm, o0, o1, o2, o3, o4, o5,
             bufA0, bufA1, zbuf, zidx, gidxT, idx0, idx1,
             gsem0, gsem1, ssem0, ssem1):
    cid = lax.axis_index("c")
    sid = lax.axis_index("s")
    wid = sid * 2 + cid
    b = wid // 2
    half = wid % 2
    a0 = half * 256
    base_flat = b * _L

    jobs = (
        (src_hbm, 0, o0), (src_hbm, 1, o1), (src_hbm, 2, o2),
        (tgt_hbm, 0, o3), (tgt_hbm, 1, o4), (tgt_hbm, 2, o5),
    )
    bufs = (bufA0, bufA1)
    gsems = (gsem0, gsem1)
    ssems = (ssem0, ssem1)
    idxs = (idx0, idx1)
    iota16 = lax.iota(jnp.int32, 16)

    zeros16 = jnp.zeros((16,), jnp.float32)
    for r in range(16):
        for t in range(_D // 16):
            zbuf[r, pl.ds(t * 16, 16)] = zeros16
    zidx[pl.ds(0, 16)] = jnp.broadcast_to(base_flat, (16,)).astype(jnp.int32)
    gidxT[pl.ds(0, 16)] = 479 + iota16
    gidxT[pl.ds(16, 16)] = 495 + iota16

    @pl.when(half == 0)
    def _():
        for _, _l, out2d in jobs:
            pltpu.async_copy(zbuf, out2d.at[zidx], ssems[0]).wait()

    chunks = []
    for e_hbm, l, out2d in jobs:
        for g in range(_NCH):
            chunks.append((e_hbm, l, out2d, g))
    n = len(chunks)

    def gather(i):
        e_hbm, l, _, g = chunks[i]
        p = i % 2
        cols = pl.ds(l * _D, _D)
        if g == _NCH - 1:
            hs = [None, None]

            @pl.when(half == 0)
            def _():
                hs[0] = pltpu.async_copy(
                    e_hbm.at[b, pl.ds(224, _CH), cols], bufs[p], gsems[p])

            @pl.when(half == 1)
            def _():
                hs[1] = pltpu.async_copy(
                    e_hbm.at[b].at[gidxT, cols], bufs[p], gsems[p])

            return hs
        a = a0 + g * _CH
        h = pltpu.async_copy(e_hbm.at[b, pl.ds(a, _CH), cols], bufs[p], gsems[p])
        return (h, None)

    def hwait(g_):
        h0, h1 = g_
        if h1 is None:
            h0.wait()
        else:
            @pl.when(half == 0)
            def _():
                h0.wait()

            @pl.when(half == 1)
            def _():
                h1.wait()

    def scatter(i):
        _, _, out2d, g = chunks[i]
        p = i % 2
        if g == _NCH - 1:
            a = 224 + half * 255
        else:
            a = a0 + g * _CH
        obase = base_flat + a + 1
        idxs[p][pl.ds(0, 16)] = obase + iota16
        idxs[p][pl.ds(16, 16)] = obase + 16 + iota16
        return (pltpu.async_copy(bufs[p], out2d.at[idxs[p]], ssems[p]), None)

    g = [None, None]
    s = [None, None]
    g[0] = gather(0)
    for i in range(n):
        p = i % 2
        q = (i + 1) % 2
        if i + 1 < n:
            if s[q] is not None:
                hwait(s[q])
                s[q] = None
            g[q] = gather(i + 1)
        hwait(g[p])
        s[p] = scatter(i)
    for s_ in s:
        if s_ is not None:
            hwait(s_)


def kernel(elmo_src, elmo_tgt):
    mesh = plsc.VectorSubcoreMesh(core_axis_name="c", subcore_axis_name="s")
    out_struct = jax.ShapeDtypeStruct((_B * _L, _D), jnp.float32)
    kern = functools.partial(
        pl.kernel,
        out_type=[out_struct] * 6,
        mesh=mesh,
        scratch_types=[
            pltpu.VMEM((_CH, _D), jnp.float32),
            pltpu.VMEM((_CH, _D), jnp.float32),
            pltpu.VMEM((16, _D), jnp.float32),
            pltpu.VMEM((16,), jnp.int32),
            pltpu.VMEM((_CH,), jnp.int32),
            pltpu.VMEM((_CH,), jnp.int32),
            pltpu.VMEM((_CH,), jnp.int32),
            pltpu.SemaphoreType.DMA,
            pltpu.SemaphoreType.DMA,
            pltpu.SemaphoreType.DMA,
            pltpu.SemaphoreType.DMA,
        ],
    )(_sc_body)
    outs = kern(elmo_src, elmo_tgt)
    return tuple(o.reshape(_B, _L, _D) for o in outs)


# speedup vs baseline: 2.5120x; 1.0026x over previous
import functools

import jax
import jax.numpy as jnp
from jax import lax
from jax.experimental import pallas as pl
from jax.experimental.pallas import tpu as pltpu
from jax.experimental.pallas import tpu_sc as plsc

_D = 1024
_B = 16
_L = 512
_CH = 32
_NCH = 8


def _sc_body(src_hbm, tgt_hbm, o0, o1, o2, o3, o4, o5,
             bufA0, bufA1, zbuf, zidx, gidxT, idx0, idx1,
             gsem0, gsem1, ssem0, ssem1):
    cid = lax.axis_index("c")
    sid = lax.axis_index("s")
    wid = sid * 2 + cid
    b = wid // 2
    half = wid % 2
    a0 = half * 256
    base_flat = b * _L

    jobs = (
        (src_hbm, 0, o0), (src_hbm, 1, o1), (src_hbm, 2, o2),
        (tgt_hbm, 0, o3), (tgt_hbm, 1, o4), (tgt_hbm, 2, o5),
    )
    bufs = (bufA0, bufA1)
    gsems = (gsem0, gsem1)
    ssems = (ssem0, ssem1)
    idxs = (idx0, idx1)
    iota16 = lax.iota(jnp.int32, 16)

    zeros16 = jnp.zeros((16,), jnp.float32)
    for r in range(16):
        for t in range(_D // 16):
            zbuf[r, pl.ds(t * 16, 16)] = zeros16
    zidx[pl.ds(0, 16)] = jnp.broadcast_to(base_flat, (16,)).astype(jnp.int32)
    gidxT[pl.ds(0, 16)] = 479 + iota16
    gidxT[pl.ds(16, 16)] = 495 + iota16

    @pl.when(half == 0)
    def _():
        for _, _l, out2d in jobs:
            pltpu.async_copy(zbuf, out2d.at[zidx], ssems[0]).wait()

    chunks = []
    for e_hbm, l, out2d in jobs:
        for g in range(_NCH):
            chunks.append((e_hbm, l, out2d, g))
    n = len(chunks)

    def gather(i):
        e_hbm, l, _, g = chunks[i]
        p = i % 2
        cols = pl.ds(l * _D, _D)
        if g == _NCH - 1:
            hs = [None, None]

            @pl.when(half == 0)
            def _():
                hs[0] = pltpu.async_copy(
                    e_hbm.at[b, pl.ds(224, _CH), cols], bufs[p], gsems[p])

            @pl.when(half == 1)
            def _():
                hs[1] = pltpu.async_copy(
                    e_hbm.at[b].at[gidxT, cols], bufs[p], gsems[p])

            return hs
        a = a0 + g * _CH
        h = pltpu.async_copy(e_hbm.at[b, pl.ds(a, _CH), cols], bufs[p], gsems[p])
        return (h, None)

    def hwait(g_):
        h0, h1 = g_
        if h1 is None:
            h0.wait()
        else:
            @pl.when(half == 0)
            def _():
                h0.wait()

            @pl.when(half == 1)
            def _():
                h1.wait()

    def scatter(i):
        _, _, out2d, g = chunks[i]
        p = i % 2
        if g == _NCH - 1:
            a = 224 + half * 255
        else:
            a = a0 + g * _CH
        obase = base_flat + a + 1
        idxs[p][pl.ds(0, 16)] = obase + iota16
        idxs[p][pl.ds(16, 16)] = obase + 16 + iota16
        return (pltpu.async_copy(bufs[p], out2d.at[idxs[p]], ssems[p]), None)

    g = [None, None]
    s = [None, None]
    g[0] = gather(0)
    for i in range(n):
        p = i % 2
        q = (i + 1) % 2
        if i + 1 < n:
            if s[q] is not None:
                hwait(s[q])
                s[q] = None
            g[q] = gather(i + 1)
        hwait(g[p])
        s[p] = scatter(i)
    for s_ in s:
        if s_ is not None:
            hwait(s_)


def kernel(elmo_src, elmo_tgt):
    mesh = plsc.VectorSubcoreMesh(core_axis_name="c", subcore_axis_name="s")
    out_struct = jax.ShapeDtypeStruct((_B * _L, _D), jnp.float32)
    kern = functools.partial(
        pl.kernel,
        out_type=[out_struct] * 6,
        mesh=mesh,
        scratch_types=[
            pltpu.VMEM((_CH, _D), jnp.float32),
            pltpu.VMEM((_CH, _D), jnp.float32),
            pltpu.VMEM((16, _D), jnp.float32),
            pltpu.VMEM((16,), jnp.int32),
            pltpu.VMEM((_CH,), jnp.int32),
            pltpu.VMEM((_CH,), jnp.int32),
            pltpu.VMEM((_CH,), jnp.int32),
            pltpu.SemaphoreType.DMA,
            pltpu.SemaphoreType.DMA,
            pltpu.SemaphoreType.DMA,
            pltpu.SemaphoreType.DMA,
        ],
        compiler_params=pltpu.CompilerParams(use_tc_tiling_on_sc=True),
    )(_sc_body)
    outs = kern(elmo_src, elmo_tgt)
    return tuple(o.reshape(_B, _L, _D) for o in outs)


# speedup vs baseline: 4.3741x; 1.7413x over previous
import functools

import jax
import jax.numpy as jnp
from jax import lax
from jax.experimental import pallas as pl
from jax.experimental.pallas import tpu as pltpu
from jax.experimental.pallas import tpu_sc as plsc

_D = 1024
_B = 16
_L = 512
_RW = 16
_CR = 2
_NCH = _RW // _CR


def _sc_body(src_t, tgt_t, o0, o1, o2, o3, o4, o5,
             bufA0, bufA1, zbuf, zidx, ix00, ix01, ix10, ix11,
             gsem0, gsem1, ssem0, ssem1):
    cid = lax.axis_index("c")
    sid = lax.axis_index("s")
    wid = sid * 2 + cid
    rb = jnp.minimum(wid * _RW, 511 - _RW)

    jobs = (
        (src_t, 0, o0), (src_t, 1, o1), (src_t, 2, o2),
        (tgt_t, 0, o3), (tgt_t, 1, o4), (tgt_t, 2, o5),
    )
    bufs = (bufA0, bufA1)
    gsems = (gsem0, gsem1)
    ssems = (ssem0, ssem1)
    idxs = ((ix00, ix01), (ix10, ix11))
    iota16 = lax.iota(jnp.int32, 16)
    zeros16 = jnp.zeros((16,), jnp.float32)

    @pl.when(wid == 0)
    def _():
        for r in range(16):
            for t in range(_D // 16):
                zbuf[r, pl.ds(t * 16, 16)] = zeros16
        zidx[pl.ds(0, 16)] = iota16 * _L
        for _, _l, out2d in jobs:
            pltpu.async_copy(zbuf, out2d.at[zidx], ssems[0]).wait()

    chunks = []
    for e_t, l, out2d in jobs:
        for g in range(_NCH):
            chunks.append((e_t, l, out2d, g))
    n = len(chunks)

    def gather(i):
        e_t, l, _, g = chunks[i]
        p = i % 2
        r0 = rb + g * _CR
        return pltpu.async_copy(
            e_t.at[pl.ds(r0, _CR), :, pl.ds(l * _D, _D)], bufs[p], gsems[p])

    def scatter(i):
        _, _, out2d, g = chunks[i]
        p = i % 2
        r0 = rb + g * _CR
        hs = []
        for r in range(_CR):
            ix = idxs[p][r]
            ix[pl.ds(0, 16)] = iota16 * _L + (r0 + r + 1)
            hs.append(pltpu.async_copy(bufs[p].at[r], out2d.at[ix], ssems[p]))
        return hs

    g = [None, None]
    s = [None, None]
    g[0] = gather(0)
    for i in range(n):
        p = i % 2
        q = (i + 1) % 2
        if i + 1 < n:
            if s[q] is not None:
                for h in s[q]:
                    h.wait()
                s[q] = None
            g[q] = gather(i + 1)
        g[p].wait()
        s[p] = scatter(i)
    for s_ in s:
        if s_ is not None:
            for h in s_:
                h.wait()


def kernel(elmo_src, elmo_tgt):
    mesh = plsc.VectorSubcoreMesh(core_axis_name="c", subcore_axis_name="s")
    out_struct = jax.ShapeDtypeStruct((_B * _L, _D), jnp.float32)
    kern = functools.partial(
        pl.kernel,
        out_type=[out_struct] * 6,
        mesh=mesh,
        scratch_types=[
            pltpu.VMEM((_CR, _B, _D), jnp.float32),
            pltpu.VMEM((_CR, _B, _D), jnp.float32),
            pltpu.VMEM((16, _D), jnp.float32),
            pltpu.VMEM((16,), jnp.int32),
            pltpu.VMEM((16,), jnp.int32),
            pltpu.VMEM((16,), jnp.int32),
            pltpu.VMEM((16,), jnp.int32),
            pltpu.VMEM((16,), jnp.int32),
            pltpu.SemaphoreType.DMA,
            pltpu.SemaphoreType.DMA,
            pltpu.SemaphoreType.DMA,
            pltpu.SemaphoreType.DMA,
        ],
    )(_sc_body)
    src_t = jnp.transpose(elmo_src, (1, 0, 2))
    tgt_t = jnp.transpose(elmo_tgt, (1, 0, 2))
    outs = kern(src_t, tgt_t)
    return tuple(o.reshape(_B, _L, _D) for o in outs)
